# Initial kernel scaffold; baseline (speedup 1.0000x reference)
import jax
import jax.numpy as jnp
from jax.experimental import pallas as pl


def kernel(x, pos_embedding):
    raise NotImplementedError("write your pallas kernel here")


# speedup vs baseline: 23.0226x; 23.0226x over previous
import jax
import jax.numpy as jnp
from jax.experimental import pallas as pl

BATCH = 4096
SEQ = 200
H_DIM = 32
ROW = SEQ * H_DIM
BB = 256


def _bcast_body(emb_ref, out_ref):
    out_ref[...] = jnp.broadcast_to(emb_ref[...], out_ref.shape)


def kernel(x, pos_embedding):
    del x
    emb_flat = pos_embedding[:SEQ].reshape(1, ROW)
    out = pl.pallas_call(
        _bcast_body,
        grid=(BATCH // BB,),
        in_specs=[pl.BlockSpec((1, ROW), lambda i: (0, 0))],
        out_specs=pl.BlockSpec((BB, ROW), lambda i: (i, 0)),
        out_shape=jax.ShapeDtypeStruct((BATCH, ROW), jnp.float32),
    )(emb_flat)
    return out.reshape(BATCH, SEQ, H_DIM)
